# baseline (speedup 1.0000x reference)
import functools

import jax
import jax.numpy as jnp
from jax import lax
from jax.experimental import pallas as pl
from jax.experimental.pallas import tpu as pltpu
from jax.experimental.pallas import tpu_sc as plsc

D = 64
CHUNK = 128
NBUF = 4
PF = 2


def _embed_kernel(n_chunks: int, idx_hbm, table_hbm, out_hbm,
                  idx_v, rows_v, gsem, wsem):
    c = lax.axis_index("c")
    s = lax.axis_index("s")
    wid = s * 2 + c
    base = wid * n_chunks

    pltpu.sync_copy(idx_hbm.at[wid], idx_v)

    def gather(j, b):
        return pltpu.async_copy(
            table_hbm.at[idx_v.at[j]], rows_v.at[b], gsem.at[b])

    def writeback(j, b):
        return pltpu.async_copy(
            rows_v.at[b], out_hbm.at[pl.ds((base + j) * CHUNK, CHUNK)],
            wsem.at[b])

    def wait_writeback(j, b):
        pltpu.make_async_copy(
            rows_v.at[b], out_hbm.at[pl.ds((base + j) * CHUNK, CHUNK)],
            wsem.at[b]).wait()

    for b in range(PF):
        gather(b, b)

    def group(g, _):
        for b in range(NBUF):
            j = g * NBUF + b
            pb = (b + PF) % NBUF

            @pl.when(j + PF < n_chunks)
            def _():
                @pl.when(j + PF >= NBUF)
                def _():
                    wait_writeback(j + PF - NBUF, pb)
                gather(j + PF, pb)

            pltpu.make_async_copy(
                table_hbm.at[idx_v.at[j]], rows_v.at[b], gsem.at[b]).wait()
            writeback(j, b)
        return 0

    lax.fori_loop(0, n_chunks // NBUF, group, 0, unroll=False)

    for k in range(min(NBUF, n_chunks)):
        j = n_chunks - min(NBUF, n_chunks) + k
        wait_writeback(j, j % NBUF)


@jax.jit
def kernel(indices, table):
    batch, seq = indices.shape
    n = batch * seq
    info = plsc.get_sparse_core_info()
    nw = info.num_cores * info.num_subcores
    assert n % (nw * CHUNK) == 0
    n_chunks = n // (nw * CHUNK)
    assert n_chunks % NBUF == 0

    idx = indices.reshape(nw, n_chunks, CHUNK).astype(jnp.int32)
    mesh = plsc.VectorSubcoreMesh(core_axis_name="c", subcore_axis_name="s")
    out = pl.kernel(
        functools.partial(_embed_kernel, n_chunks),
        mesh=mesh,
        out_type=jax.ShapeDtypeStruct((n, D), jnp.float32),
        compiler_params=pltpu.CompilerParams(use_tc_tiling_on_sc=False),
        scratch_types=[
            pltpu.VMEM((n_chunks, CHUNK), jnp.int32),
            pltpu.VMEM((NBUF, CHUNK, D), jnp.float32),
            pltpu.SemaphoreType.DMA((NBUF,)),
            pltpu.SemaphoreType.DMA((NBUF,)),
        ],
    )(idx, table)
    return out.reshape(batch, seq, D)
